# baseline (device time: 14135 ns/iter reference)
import jax
import jax.numpy as jnp
from jax import lax
from jax.experimental import pallas as pl
from jax.experimental.pallas import tpu as pltpu

N_DEV = 8
B = 128


def kernel(x, w_mat):
    k, n_per = x.shape
    k_w, n = w_mat.shape

    def body(x_ref, w_ref, out_ref, gather_ref, send_sems, recv_sems):
        my_pos = lax.axis_index("i")

        barrier_sem = pltpu.get_barrier_semaphore()
        for d in range(1, N_DEV):
            peer = lax.rem(my_pos + d, N_DEV)
            pl.semaphore_signal(
                barrier_sem, inc=1,
                device_id=(peer,), device_id_type=pl.DeviceIdType.MESH,
            )
        pl.semaphore_wait(barrier_sem, N_DEV - 1)

        sends = []
        for d in range(1, N_DEV):
            r = lax.rem(my_pos + d, N_DEV)
            rdma = pltpu.make_async_remote_copy(
                src_ref=x_ref.at[pl.ds(r * B, B), :],
                dst_ref=gather_ref.at[my_pos],
                send_sem=send_sems.at[d],
                recv_sem=recv_sems.at[my_pos],
                device_id=(r,),
                device_id_type=pl.DeviceIdType.MESH,
            )
            rdma.start()
            sends.append(rdma)

        gather_ref[my_pos] = x_ref[pl.ds(my_pos * B, B), :]

        for j in range(N_DEV):
            @pl.when(j != my_pos)
            def _():
                recv = pltpu.make_async_remote_copy(
                    src_ref=gather_ref.at[j],
                    dst_ref=gather_ref.at[j],
                    send_sem=send_sems.at[0],
                    recv_sem=recv_sems.at[j],
                    device_id=(my_pos,),
                    device_id_type=pl.DeviceIdType.MESH,
                )
                recv.wait_recv()

        for rdma in sends:
            rdma.wait_send()

        y = jnp.zeros((B, n), dtype=jnp.float32)
        for j in range(N_DEV):
            y = y + jnp.dot(
                gather_ref[j],
                w_ref[j * B:(j + 1) * B, :],
                preferred_element_type=jnp.float32,
            )
        out_ref[:, :] = y * (1.0 / (1.0 + jnp.exp(-y)))

    return pl.pallas_call(
        body,
        out_shape=jax.ShapeDtypeStruct((B, n), jnp.float32),
        in_specs=[
            pl.BlockSpec(memory_space=pltpu.VMEM),
            pl.BlockSpec(memory_space=pltpu.VMEM),
        ],
        out_specs=pl.BlockSpec(memory_space=pltpu.VMEM),
        scratch_shapes=[
            pltpu.VMEM((N_DEV, B, n_per), jnp.float32),
            pltpu.SemaphoreType.DMA((N_DEV,)),
            pltpu.SemaphoreType.DMA((N_DEV,)),
        ],
        compiler_params=pltpu.CompilerParams(collective_id=0),
    )(x, w_mat)
